# baseline (device time: 168089 ns/iter reference)
import jax
import jax.numpy as jnp
from jax import lax
from jax.experimental import pallas as pl
from jax.experimental.pallas import tpu as pltpu

N_DEV = 16
B, SQ, D = 4, 256, 1024
H_LOC, DH = 8, 128
ROWS = B * SQ
CHUNK = ROWS // N_DEV
SCALE = 0.08838834764831843


def kernel(x, Wq, Wo, Wk, Wv):
    x2 = x.reshape(ROWS, D)

    def body(x_ref, wq_ref, wk_ref, wv_ref, wo_ref, out_ref,
             q_ref, k_ref, v_ref, o_ref, acc_ref, comm_ref,
             send_sems, recv_sems):
        my = lax.axis_index("i")
        left = (my + N_DEV - 1) % N_DEV
        right = (my + 1) % N_DEV

        q_ref[...] = jnp.dot(x_ref[...], wq_ref[...],
                             preferred_element_type=jnp.float32)
        k_ref[...] = jnp.dot(x_ref[...], wk_ref[...],
                             preferred_element_type=jnp.float32)
        v_ref[...] = jnp.dot(x_ref[...], wv_ref[...],
                             preferred_element_type=jnp.float32)

        for b in range(B):
            r0 = b * SQ
            for h in range(H_LOC):
                c0 = h * DH
                q = q_ref[r0:r0 + SQ, c0:c0 + DH]
                k = k_ref[r0:r0 + SQ, c0:c0 + DH]
                v = v_ref[r0:r0 + SQ, c0:c0 + DH]
                s = lax.dot_general(q, k, (((1,), (1,)), ((), ())),
                                    preferred_element_type=jnp.float32) * SCALE
                m = jnp.max(s, axis=1, keepdims=True)
                p = jnp.exp(s - m)
                l = jnp.sum(p, axis=1, keepdims=True)
                o_ref[r0:r0 + SQ, c0:c0 + DH] = jnp.dot(
                    p / l, v, preferred_element_type=jnp.float32)

        acc_ref[...] = jnp.dot(o_ref[...], wo_ref[...],
                               preferred_element_type=jnp.float32)

        barrier = pltpu.get_barrier_semaphore()
        for nbr in (left, right):
            pl.semaphore_signal(barrier, inc=1, device_id=(nbr,),
                                device_id_type=pl.DeviceIdType.MESH)
        pl.semaphore_wait(barrier, 2)

        comm_ref[0, :, :] = acc_ref[pl.ds(my * CHUNK, CHUNK), :]
        for t in range(2 * (N_DEV - 1)):
            send_slot = t % 2
            recv_slot = (t + 1) % 2
            rdma = pltpu.make_async_remote_copy(
                src_ref=comm_ref.at[send_slot],
                dst_ref=comm_ref.at[recv_slot],
                send_sem=send_sems.at[send_slot],
                recv_sem=recv_sems.at[recv_slot],
                device_id=(right,),
                device_id_type=pl.DeviceIdType.MESH,
            )
            rdma.start()
            rdma.wait()
            c = (my + 2 * N_DEV - t - 1) % N_DEV
            if t < N_DEV - 1:
                comm_ref[recv_slot, :, :] = (
                    comm_ref[recv_slot, :, :]
                    + acc_ref[pl.ds(c * CHUNK, CHUNK), :])
                if t == N_DEV - 2:
                    out_ref[pl.ds(c * CHUNK, CHUNK), :] = comm_ref[recv_slot, :, :]
            else:
                out_ref[pl.ds(c * CHUNK, CHUNK), :] = comm_ref[recv_slot, :, :]

    out = pl.pallas_call(
        body,
        out_shape=jax.ShapeDtypeStruct((ROWS, D), jnp.float32),
        in_specs=[pl.BlockSpec(memory_space=pltpu.VMEM)] * 5,
        out_specs=pl.BlockSpec(memory_space=pltpu.VMEM),
        scratch_shapes=[
            pltpu.VMEM((ROWS, D), jnp.float32),
            pltpu.VMEM((ROWS, D), jnp.float32),
            pltpu.VMEM((ROWS, D), jnp.float32),
            pltpu.VMEM((ROWS, D), jnp.float32),
            pltpu.VMEM((ROWS, D), jnp.float32),
            pltpu.VMEM((2, CHUNK, D), jnp.float32),
            pltpu.SemaphoreType.DMA((2,)),
            pltpu.SemaphoreType.DMA((2,)),
        ],
        compiler_params=pltpu.CompilerParams(collective_id=0),
    )(x2, Wq, Wk, Wv, Wo)
    return out.reshape(B, SQ, D)


# device time: 73365 ns/iter; 2.2911x vs baseline; 2.2911x over previous
import jax
import jax.numpy as jnp
from jax import lax
from jax.experimental import pallas as pl
from jax.experimental.pallas import tpu as pltpu

N_DEV = 16
B, SQ, D = 4, 256, 1024
H_LOC, DH = 8, 128
ROWS = B * SQ
SCALE = 0.08838834764831843

MASKS = (
    (1, 3, 4, 8),
    (4, 8, 1, 3),
)
CW = D // 2
LENS = (512, 256, 128, 64, 64, 128, 256, 512)
OFFS = (0, 512, 768, 896, 960, 1024, 1152, 1408)
BUF_ROWS = 1920


def kernel(x, Wq, Wo, Wk, Wv):
    x2 = x.reshape(ROWS, D)

    def body(x_ref, wq_ref, wk_ref, wv_ref, wo_ref, out_ref,
             q_ref, k_ref, v_ref, o_ref, acc_ref,
             send_buf, recv_buf, send_sems, recv_sems):
        my = lax.axis_index("i")
        i0 = my & 1
        i1 = (my >> 1) & 1
        i2 = (my >> 2) & 1
        i3 = (my >> 3) & 1
        bits = {1: i0 ^ i1, 3: i1, 4: i2, 8: i3}
        partner = {m: my ^ m for m in (1, 3, 4, 8)}

        q_ref[...] = jnp.dot(x_ref[...], wq_ref[...],
                             preferred_element_type=jnp.float32)
        k_ref[...] = jnp.dot(x_ref[...], wk_ref[...],
                             preferred_element_type=jnp.float32)
        v_ref[...] = jnp.dot(x_ref[...], wv_ref[...],
                             preferred_element_type=jnp.float32)

        for b in range(B):
            r0 = b * SQ
            for h in range(H_LOC):
                c0 = h * DH
                q = q_ref[r0:r0 + SQ, c0:c0 + DH]
                k = k_ref[r0:r0 + SQ, c0:c0 + DH]
                v = v_ref[r0:r0 + SQ, c0:c0 + DH]
                s = lax.dot_general(q, k, (((1,), (1,)), ((), ())),
                                    preferred_element_type=jnp.float32) * SCALE
                m = jnp.max(s, axis=1, keepdims=True)
                p = jnp.exp(s - m)
                l = jnp.sum(p, axis=1, keepdims=True)
                o_ref[r0:r0 + SQ, c0:c0 + DH] = jnp.dot(
                    p / l, v, preferred_element_type=jnp.float32)

        acc_ref[...] = jnp.dot(o_ref[...], wo_ref[...],
                               preferred_element_type=jnp.float32)

        barrier = pltpu.get_barrier_semaphore()
        for m in (1, 3, 4, 8):
            pl.semaphore_signal(barrier, inc=1, device_id=(partner[m],),
                                device_id_type=pl.DeviceIdType.MESH)
        pl.semaphore_wait(barrier, 4)

        def start_exchange(st, s, m, src_rows_ref):
            L = LENS[s]
            send_buf[st, OFFS[s]:OFFS[s] + L, :] = src_rows_ref.astype(
                jnp.bfloat16)
            rdma = pltpu.make_async_remote_copy(
                src_ref=send_buf.at[st, pl.ds(OFFS[s], L), :],
                dst_ref=recv_buf.at[st, pl.ds(OFFS[s], L), :],
                send_sem=send_sems.at[st * 8 + s],
                recv_sem=recv_sems.at[st * 8 + s],
                device_id=(partner[m],),
                device_id_type=pl.DeviceIdType.MESH,
            )
            rdma.start()
            return rdma

        lo = [jnp.int32(0), jnp.int32(0)]
        for s in range(4):
            half = ROWS >> (s + 1)
            rdmas = []
            for st in range(2):
                m = MASKS[st][s]
                send_lo = lo[st] + (1 - bits[m]) * half
                c0 = st * CW
                rdmas.append(start_exchange(
                    st, s, m, acc_ref[pl.ds(send_lo, half), c0:c0 + CW]))
            for st in range(2):
                m = MASKS[st][s]
                rdmas[st].wait()
                keep_lo = lo[st] + bits[m] * half
                c0 = st * CW
                acc_ref[pl.ds(keep_lo, half), c0:c0 + CW] = (
                    acc_ref[pl.ds(keep_lo, half), c0:c0 + CW]
                    + recv_buf[st, OFFS[s]:OFFS[s] + half, :].astype(
                        jnp.float32))
                lo[st] = keep_lo

        for st in range(2):
            c0 = st * CW
            out_ref[pl.ds(lo[st], 64), c0:c0 + CW] = (
                acc_ref[pl.ds(lo[st], 64), c0:c0 + CW])

        for s in range(4, 8):
            L = LENS[s]
            rdmas = []
            for st in range(2):
                m = MASKS[st][7 - s]
                c0 = st * CW
                rdmas.append(start_exchange(
                    st, s, m, out_ref[pl.ds(lo[st], L), c0:c0 + CW]))
            for st in range(2):
                m = MASKS[st][7 - s]
                rdmas[st].wait()
                b = bits[m]
                c0 = st * CW
                recv_lo = lo[st] + (1 - 2 * b) * L
                out_ref[pl.ds(recv_lo, L), c0:c0 + CW] = (
                    recv_buf[st, OFFS[s]:OFFS[s] + L, :].astype(jnp.float32))
                lo[st] = lo[st] - b * L

    out = pl.pallas_call(
        body,
        out_shape=jax.ShapeDtypeStruct((ROWS, D), jnp.float32),
        in_specs=[pl.BlockSpec(memory_space=pltpu.VMEM)] * 5,
        out_specs=pl.BlockSpec(memory_space=pltpu.VMEM),
        scratch_shapes=[
            pltpu.VMEM((ROWS, D), jnp.float32),
            pltpu.VMEM((ROWS, D), jnp.float32),
            pltpu.VMEM((ROWS, D), jnp.float32),
            pltpu.VMEM((ROWS, D), jnp.float32),
            pltpu.VMEM((ROWS, D), jnp.float32),
            pltpu.VMEM((2, BUF_ROWS, CW), jnp.bfloat16),
            pltpu.VMEM((2, BUF_ROWS, CW), jnp.bfloat16),
            pltpu.SemaphoreType.DMA((16,)),
            pltpu.SemaphoreType.DMA((16,)),
        ],
        compiler_params=pltpu.CompilerParams(collective_id=0),
    )(x2, Wq, Wk, Wv, Wo)
    return out.reshape(B, SQ, D)
